# baseline (speedup 1.0000x reference)
import functools

import jax
import jax.numpy as jnp
from jax import lax
from jax.experimental import pallas as pl
from jax.experimental.pallas import tpu as pltpu
from jax.experimental.pallas import tpu_sc as plsc

B, Q, C, CODE = 8, 5000, 10, 10
K = 300
N = Q * C
M = 512
NPAD = 50048
CHUNK = NPAD // 4
CAP_W = 192
CAP = 4 * CAP_W
LOCBUF = CAP_W + 16


def _keys(f32val):
    b32 = lax.bitcast_convert_type(f32val, jnp.int32)
    return jnp.where(b32 < 0, b32 ^ jnp.int32(0x7FFFFFFF), b32)


WIN_LO, WIN_HI = 384, 560


def _thresh_body(scores_ref, thr_ref, slo, shi, st, sf):
    key = _keys(scores_ref[...])
    lo0 = jnp.min(key, axis=1, keepdims=True)
    hi0 = jnp.max(key, axis=1, keepdims=True) + 1
    slo[...] = jnp.broadcast_to(lo0, (B, 128))
    shi[...] = jnp.broadcast_to(hi0, (B, 128))
    st[...] = jnp.broadcast_to(lo0, (B, 128))
    sf[...] = jnp.zeros((B, 128), jnp.int32)

    def cond(carry):
        it, af = carry
        return (it < 32) & ~af

    def body(carry):
        it, _ = carry
        lo = slo[...]
        hi = shi[...]
        mid = (lo >> 1) + (hi >> 1) + (lo & hi & 1)
        cnt1 = jnp.sum((key >= mid[:, 0:1]).astype(jnp.int32),
                       axis=1, keepdims=True)
        cnt = jnp.broadcast_to(cnt1, (B, 128))
        fnd = sf[...]
        ok = ((cnt >= WIN_LO) & (cnt <= WIN_HI)).astype(jnp.int32)
        st[...] = jnp.where((ok > 0) & (fnd == 0), mid, st[...])
        fnd = fnd | ok
        sf[...] = fnd
        pred = cnt >= M
        slo[...] = jnp.where(pred, mid, lo)
        shi[...] = jnp.where(pred, hi, mid)
        return it + 1, jnp.all(fnd > 0)

    lax.while_loop(cond, body, (jnp.int32(0), False))
    t = jnp.where(sf[...] > 0, st[...], slo[...])
    thr_ref[...] = t[:, 0:16]


def _stage_a(flat_scores):
    return pl.pallas_call(
        _thresh_body,
        out_shape=jax.ShapeDtypeStruct((B, 16), jnp.int32),
        scratch_shapes=[pltpu.VMEM((B, 128), jnp.int32) for _ in range(4)],
    )(flat_scores)


def _stage_b(scores_pad_flat, thr_flat):
    mesh = plsc.VectorSubcoreMesh(core_axis_name="c", subcore_axis_name="s")

    @functools.partial(
        pl.kernel,
        mesh=mesh,
        compiler_params=pltpu.CompilerParams(needs_layout_passes=False),
        out_type=[
            jax.ShapeDtypeStruct((B * CAP,), jnp.float32),
            jax.ShapeDtypeStruct((B * CAP,), jnp.int32),
        ],
        scratch_types=[
            pltpu.VMEM((CHUNK,), jnp.float32),
            pltpu.VMEM((LOCBUF,), jnp.float32),
            pltpu.VMEM((LOCBUF,), jnp.int32),
            pltpu.VMEM((16,), jnp.int32),
        ],
    )
    def sc_kernel(scores_hbm, thr_hbm, out_val_hbm, out_idx_hbm,
                  chunk_v, loc_val, loc_idx, thr_v):
        wid = lax.axis_index("s") * 2 + lax.axis_index("c")
        b = wid // 4
        q = wid % 4

        pltpu.sync_copy(scores_hbm.at[pl.ds(b * NPAD + q * CHUNK, CHUNK)],
                        chunk_v)
        pltpu.sync_copy(thr_hbm.at[pl.ds(b * 16, 16)], thr_v)
        tvec = thr_v[...]

        neg = jnp.full((16,), -1e30, jnp.float32)
        zero = jnp.zeros((16,), jnp.int32)
        for j in range(LOCBUF // 16):
            loc_val[pl.ds(j * 16, 16)] = neg
            loc_idx[pl.ds(j * 16, 16)] = zero

        iota16 = lax.iota(jnp.int32, 16)
        chunk_base = q * CHUNK

        def body(i, off):
            v = chunk_v[pl.ds(i * 16, 16)]
            key = _keys(v)
            m = key >= tvec
            cnt = plsc.all_reduce_population_count(m)[0]
            offc = jnp.minimum(off, CAP_W)
            plsc.store_compressed(loc_val.at[pl.ds(offc, 16)], v, mask=m)
            plsc.store_compressed(loc_idx.at[pl.ds(offc, 16)],
                                  iota16 + (chunk_base + i * 16), mask=m)
            return jnp.minimum(off + cnt, CAP_W)

        lax.fori_loop(0, CHUNK // 16, body, jnp.int32(0))

        dst = b * CAP + q * CAP_W
        pltpu.sync_copy(loc_val.at[pl.ds(0, CAP_W)],
                        out_val_hbm.at[pl.ds(dst, CAP_W)])
        pltpu.sync_copy(loc_idx.at[pl.ds(0, CAP_W)],
                        out_idx_hbm.at[pl.ds(dst, CAP_W)])

    return sc_kernel(scores_pad_flat, thr_flat)


def _rank_body(probs_ref, idx_ref, bbox_ref, boxes_ref, scores_ref, labels_ref):
    p = probs_ref[0, 0, :]
    ii = idx_ref[0, 0, :]

    pj = p[:, None]
    pi = p[None, :]
    beats = (pj > pi) | ((pj == pi) & (ii[:, None] < ii[None, :]))
    ranks = jnp.sum(beats.astype(jnp.int32), axis=0)

    sel = (lax.broadcasted_iota(jnp.int32, (K, CAP), 0) == ranks[None, :])
    self_f = sel.astype(jnp.float32)
    scores = jnp.sum(self_f * p[None, :], axis=1)
    sel_idx = jnp.sum(self_f * ii.astype(jnp.float32)[None, :],
                      axis=1).astype(jnp.int32)

    labels = sel_idx % C
    queries = sel_idx // C
    q1 = queries // 125
    q2 = queries % 125

    onehot1 = (q1[:, None] ==
               lax.broadcasted_iota(jnp.int32, (K, 40), 1)).astype(jnp.float32)
    rows = jnp.dot(onehot1, bbox_ref[0], preferred_element_type=jnp.float32)
    onehot2 = (q2[:, None] ==
               lax.broadcasted_iota(jnp.int32, (K, 125), 1)).astype(jnp.float32)

    def pick(c):
        return jnp.sum(rows[:, c * 125:(c + 1) * 125] * onehot2, axis=1)

    x = pick(0)
    y = pick(1)
    z = pick(2)
    w = pick(3)
    l = pick(4)
    h = pick(5)
    yaw = jnp.arctan2(pick(6), pick(7))
    vx = pick(8)
    vy = pick(9)
    vz = jnp.zeros_like(x)
    boxes = jnp.stack([x, y, z, w, l, h, yaw, vx, vy, vz], axis=-1)

    boxes_ref[0] = boxes
    scores_ref[0, 0, :] = scores
    labels_ref[0, 0, :] = labels


def _stage_c(cand_probs, cand_idx, bbox_preds):
    return pl.pallas_call(
        _rank_body,
        grid=(B,),
        in_specs=[
            pl.BlockSpec((1, 1, CAP), lambda b: (b, 0, 0)),
            pl.BlockSpec((1, 1, CAP), lambda b: (b, 0, 0)),
            pl.BlockSpec((1, 40, 1250), lambda b: (b, 0, 0)),
        ],
        out_specs=[
            pl.BlockSpec((1, K, CODE), lambda b: (b, 0, 0)),
            pl.BlockSpec((1, 1, K), lambda b: (b, 0, 0)),
            pl.BlockSpec((1, 1, K), lambda b: (b, 0, 0)),
        ],
        out_shape=[
            jax.ShapeDtypeStruct((B, K, CODE), jnp.float32),
            jax.ShapeDtypeStruct((B, 1, K), jnp.float32),
            jax.ShapeDtypeStruct((B, 1, K), jnp.int32),
        ],
    )(cand_probs, cand_idx, bbox_preds)


def kernel(cls_scores, bbox_preds):
    flat = cls_scores.reshape(B, N)
    thr = _stage_a(flat)

    pad = jnp.pad(flat, ((0, 0), (0, NPAD - N)),
                  constant_values=-jnp.inf).reshape(-1)
    cand_val_flat, cand_idx_flat = _stage_b(pad, thr.reshape(-1))

    cand_logit = cand_val_flat.reshape(B, 1, CAP)
    cand_probs = jax.nn.sigmoid(cand_logit)
    cand_idx = cand_idx_flat.reshape(B, 1, CAP)

    bbox_r = jnp.transpose(bbox_preds.reshape(B, 40, 125, CODE),
                           (0, 1, 3, 2)).reshape(B, 40, 1250)
    boxes, scores, labels = _stage_c(cand_probs, cand_idx, bbox_r)
    return boxes, scores.reshape(B, K), labels.reshape(B, K)


# speedup vs baseline: 1.1264x; 1.1264x over previous
import functools

import jax
import jax.numpy as jnp
from jax import lax
from jax.experimental import pallas as pl
from jax.experimental.pallas import tpu as pltpu
from jax.experimental.pallas import tpu_sc as plsc

B, Q, C, CODE = 8, 5000, 10, 10
K = 300
N = Q * C
M = 512
NPAD = 50048
CHUNK = NPAD // 4
CAP_W = 192
CAP = 4 * CAP_W
LOCBUF = CAP_W + 16


def _keys(f32val):
    b32 = lax.bitcast_convert_type(f32val, jnp.int32)
    return jnp.where(b32 < 0, b32 ^ jnp.int32(0x7FFFFFFF), b32)


WIN_LO, WIN_HI = 384, 560


def _thresh_body(scores_ref, thr_ref, slo, shi, st, sf):
    key = _keys(scores_ref[...])
    lo0 = jnp.min(key, axis=1, keepdims=True)
    hi0 = jnp.max(key, axis=1, keepdims=True) + 1
    slo[...] = jnp.broadcast_to(lo0, (B, 128))
    shi[...] = jnp.broadcast_to(hi0, (B, 128))
    st[...] = jnp.broadcast_to(lo0, (B, 128))
    sf[...] = jnp.zeros((B, 128), jnp.int32)

    def cond(carry):
        it, af = carry
        return (it < 32) & ~af

    def body(carry):
        it, _ = carry
        lo = slo[...]
        hi = shi[...]
        mid = (lo >> 1) + (hi >> 1) + (lo & hi & 1)
        cnt1 = jnp.sum((key >= mid[:, 0:1]).astype(jnp.int32),
                       axis=1, keepdims=True)
        cnt = jnp.broadcast_to(cnt1, (B, 128))
        fnd = sf[...]
        ok = ((cnt >= WIN_LO) & (cnt <= WIN_HI)).astype(jnp.int32)
        st[...] = jnp.where((ok > 0) & (fnd == 0), mid, st[...])
        fnd = fnd | ok
        sf[...] = fnd
        pred = cnt >= M
        slo[...] = jnp.where(pred, mid, lo)
        shi[...] = jnp.where(pred, hi, mid)
        return it + 1, jnp.all(fnd > 0)

    lax.while_loop(cond, body, (jnp.int32(0), False))
    t = jnp.where(sf[...] > 0, st[...], slo[...])
    thr_ref[...] = t[:, 0:16]


def _stage_a(flat_scores):
    return pl.pallas_call(
        _thresh_body,
        out_shape=jax.ShapeDtypeStruct((B, 16), jnp.int32),
        scratch_shapes=[pltpu.VMEM((B, 128), jnp.int32) for _ in range(4)],
    )(flat_scores)


TAIL = N - 3 * CHUNK


def _stage_b(flat_scores, thr_flat):
    mesh = plsc.VectorSubcoreMesh(core_axis_name="c", subcore_axis_name="s")

    @functools.partial(
        pl.kernel,
        mesh=mesh,
        compiler_params=pltpu.CompilerParams(needs_layout_passes=False),
        out_type=[
            jax.ShapeDtypeStruct((B * CAP,), jnp.float32),
            jax.ShapeDtypeStruct((B * CAP,), jnp.int32),
        ],
        scratch_types=[
            pltpu.VMEM((CHUNK,), jnp.float32),
            pltpu.VMEM((LOCBUF,), jnp.float32),
            pltpu.VMEM((LOCBUF,), jnp.int32),
            pltpu.VMEM((16,), jnp.int32),
        ],
    )
    def sc_kernel(scores_hbm, thr_hbm, out_val_hbm, out_idx_hbm,
                  chunk_v, loc_val, loc_idx, thr_v):
        wid = lax.axis_index("s") * 2 + lax.axis_index("c")
        b = wid // 4
        q = wid % 4

        pltpu.sync_copy(scores_hbm.at[pl.ds(b * NPAD + q * CHUNK, CHUNK)],
                        chunk_v)
        pltpu.sync_copy(thr_hbm.at[pl.ds(b * 16, 16)], thr_v)
        tvec = thr_v[...]

        neg = jnp.full((16,), -1e30, jnp.float32)
        zero = jnp.zeros((16,), jnp.int32)
        for j in range(LOCBUF // 16):
            loc_val[pl.ds(j * 16, 16)] = neg
            loc_idx[pl.ds(j * 16, 16)] = zero

        iota16 = lax.iota(jnp.int32, 16)
        chunk_base = q * CHUNK

        def body(i, off):
            v = chunk_v[pl.ds(i * 16, 16)]
            key = _keys(v)
            m = key >= tvec
            cnt = plsc.all_reduce_population_count(m)[0]
            offc = jnp.minimum(off, CAP_W)
            plsc.store_compressed(loc_val.at[pl.ds(offc, 16)], v, mask=m)
            plsc.store_compressed(loc_idx.at[pl.ds(offc, 16)],
                                  iota16 + (chunk_base + i * 16), mask=m)
            return jnp.minimum(off + cnt, CAP_W)

        lax.fori_loop(0, CHUNK // 16, body, jnp.int32(0))

        dst = b * CAP + q * CAP_W
        pltpu.sync_copy(loc_val.at[pl.ds(0, CAP_W)],
                        out_val_hbm.at[pl.ds(dst, CAP_W)])
        pltpu.sync_copy(loc_idx.at[pl.ds(0, CAP_W)],
                        out_idx_hbm.at[pl.ds(dst, CAP_W)])

    return sc_kernel(flat_scores, thr_flat)


def _rank_body(probs_ref, idx_ref, bbox_ref, boxes_ref, scores_ref, labels_ref):
    p = probs_ref[0, 0, :]
    ii = idx_ref[0, 0, :]

    pj = p[:, None]
    pi = p[None, :]
    beats = (pj > pi) | ((pj == pi) & (ii[:, None] < ii[None, :]))
    ranks = jnp.sum(beats.astype(jnp.int32), axis=0)

    sel = (lax.broadcasted_iota(jnp.int32, (K, CAP), 0) == ranks[None, :])
    self_f = sel.astype(jnp.float32)
    scores = jnp.sum(self_f * p[None, :], axis=1)
    sel_idx = jnp.sum(self_f * ii.astype(jnp.float32)[None, :],
                      axis=1).astype(jnp.int32)

    labels = sel_idx % C
    queries = sel_idx // C

    onehot = (queries[:, None] ==
              lax.broadcasted_iota(jnp.int32, (K, Q), 1)).astype(jnp.float32)
    selrows = jnp.dot(onehot, bbox_ref[0], preferred_element_type=jnp.float32)

    x = selrows[:, 0]
    y = selrows[:, 1]
    z = selrows[:, 2]
    w = selrows[:, 3]
    l = selrows[:, 4]
    h = selrows[:, 5]
    yaw = jnp.arctan2(selrows[:, 6], selrows[:, 7])
    vx = selrows[:, 8]
    vy = selrows[:, 9]
    vz = jnp.zeros_like(x)
    boxes = jnp.stack([x, y, z, w, l, h, yaw, vx, vy, vz], axis=-1)

    boxes_ref[0] = boxes
    scores_ref[0, 0, :] = scores
    labels_ref[0, 0, :] = labels


def _stage_c(cand_probs, cand_idx, bbox_preds):
    return pl.pallas_call(
        _rank_body,
        grid=(B,),
        in_specs=[
            pl.BlockSpec((1, 1, CAP), lambda b: (b, 0, 0)),
            pl.BlockSpec((1, 1, CAP), lambda b: (b, 0, 0)),
            pl.BlockSpec((1, Q, CODE), lambda b: (b, 0, 0)),
        ],
        out_specs=[
            pl.BlockSpec((1, K, CODE), lambda b: (b, 0, 0)),
            pl.BlockSpec((1, 1, K), lambda b: (b, 0, 0)),
            pl.BlockSpec((1, 1, K), lambda b: (b, 0, 0)),
        ],
        out_shape=[
            jax.ShapeDtypeStruct((B, K, CODE), jnp.float32),
            jax.ShapeDtypeStruct((B, 1, K), jnp.float32),
            jax.ShapeDtypeStruct((B, 1, K), jnp.int32),
        ],
    )(cand_probs, cand_idx, bbox_preds)


def kernel(cls_scores, bbox_preds):
    flat = cls_scores.reshape(B, N)
    thr = _stage_a(flat)

    pad = jnp.pad(flat, ((0, 0), (0, NPAD - N)),
                  constant_values=-jnp.inf).reshape(-1)
    cand_val_flat, cand_idx_flat = _stage_b(pad, thr.reshape(-1))

    cand_logit = cand_val_flat.reshape(B, 1, CAP)
    cand_probs = jax.nn.sigmoid(cand_logit)
    cand_idx = cand_idx_flat.reshape(B, 1, CAP)

    boxes, scores, labels = _stage_c(cand_probs, cand_idx, bbox_preds)
    return boxes, scores.reshape(B, K), labels.reshape(B, K)
